# baseline (speedup 1.0000x reference)
import functools

import jax
import jax.numpy as jnp
from jax import lax
from jax.experimental import pallas as pl
from jax.experimental.pallas import tpu as pltpu
from jax.experimental.pallas import tpu_sc as plsc

D = 768
NTOK = 2048
BATCH = 4
B = BATCH * NTOK

NC = 2
NS = 16
NW = NC * NS
PPW = NTOK // NW
C = 32
NCHUNK = BATCH * PPW // C
QPB = PPW // C
NBUF = 4

_mesh = plsc.VectorSubcoreMesh(core_axis_name="c", subcore_axis_name="s")


@functools.partial(
    pl.kernel,
    mesh=_mesh,
    out_type=jax.ShapeDtypeStruct((B, D), jnp.float32),
    scratch_types=[
        pltpu.VMEM((BATCH * PPW,), jnp.int32),
    ] + [pltpu.VMEM((C, D), jnp.float32) for _ in range(NBUF)] + [
        pltpu.SemaphoreType.DMA,
        pltpu.SemaphoreType.DMA,
        pltpu.SemaphoreType.DMA,
    ],
)
def _emb_kernel(idx_hbm, table_hbm, pe_hbm, out_hbm,
                idx_v, b0, b1, b2, b3, gsem, psem, osem):
    wid = lax.axis_index("s") * NC + lax.axis_index("c")
    p0 = wid * PPW
    for b in range(BATCH):
        pltpu.sync_copy(idx_hbm.at[pl.ds(b * NTOK + p0, PPW)],
                        idx_v.at[pl.ds(b * PPW, PPW)])
    bufs = [b0, b1, b2, b3]

    def pe_rows(ci):
        return pl.ds(p0 + (ci % QPB) * C, C)

    ph = [pltpu.async_copy(pe_hbm.at[pe_rows(ci), :], bufs[ci % NBUF], psem)
          for ci in range(2)]
    gh = []
    oh = []
    for ci in range(NCHUNK):
        if ci + 2 < NCHUNK:
            if ci + 2 - NBUF >= 0:
                oh[ci + 2 - NBUF].wait()
            ph.append(pltpu.async_copy(
                pe_hbm.at[pe_rows(ci + 2), :], bufs[(ci + 2) % NBUF], psem))
        if ci == 0:
            ph[0].wait()
            gh.append(pltpu.async_copy(
                table_hbm.at[idx_v.at[pl.ds(0, C)]], bufs[0], gsem, add=True))
        if ci + 1 < NCHUNK:
            ph[ci + 1].wait()
            gh.append(pltpu.async_copy(
                table_hbm.at[idx_v.at[pl.ds((ci + 1) * C, C)]],
                bufs[(ci + 1) % NBUF], gsem, add=True))
        gh[ci].wait()
        b = ci // QPB
        oh.append(pltpu.async_copy(
            bufs[ci % NBUF],
            out_hbm.at[pl.ds(b * NTOK + p0 + (ci % QPB) * C, C), :], osem))
    for hh in oh[NCHUNK - NBUF + 2:]:
        hh.wait()


def kernel(x, embed_weight, positional_encoding):
    idx = x.reshape(-1).astype(jnp.int32)
    out = _emb_kernel(idx, embed_weight, positional_encoding)
    return out.reshape(x.shape[0], x.shape[1], D)


# speedup vs baseline: 1.1298x; 1.1298x over previous
import functools

import jax
import jax.numpy as jnp
from jax import lax
from jax.experimental import pallas as pl
from jax.experimental.pallas import tpu as pltpu
from jax.experimental.pallas import tpu_sc as plsc

D = 768
NTOK = 2048
BATCH = 4
B = BATCH * NTOK

NC = 2
NS = 16
NW = NC * NS
PPW = NTOK // NW
C = 32
NCHUNK = BATCH * PPW // C
QPB = PPW // C
NBUF = 3

_mesh = plsc.VectorSubcoreMesh(core_axis_name="c", subcore_axis_name="s")


@functools.partial(
    pl.kernel,
    mesh=_mesh,
    out_type=jax.ShapeDtypeStruct((B, D), jnp.float32),
    scratch_types=[
        pltpu.VMEM((BATCH * PPW,), jnp.int32),
        pltpu.VMEM_SHARED((NS * PPW, D), jnp.float32),
    ] + [pltpu.VMEM((C, D), jnp.float32) for _ in range(NBUF)] + [
        pltpu.SemaphoreType.DMA,
        pltpu.SemaphoreType.DMA,
        pltpu.SemaphoreType.DMA,
        pltpu.SemaphoreType.DMA,
    ],
)
def _emb_kernel(idx_hbm, table_hbm, pe_hbm, out_hbm,
                idx_v, pbuf, b0, b1, b2, gsem, psem, osem, isem):
    sid = lax.axis_index("s")
    wid = sid * NC + lax.axis_index("c")
    p0 = wid * PPW
    sbase = sid * PPW
    ih = [pltpu.async_copy(idx_hbm.at[pl.ds(b * NTOK + p0, PPW)],
                           idx_v.at[pl.ds(b * PPW, PPW)], isem)
          for b in range(BATCH)]
    pltpu.sync_copy(pe_hbm.at[pl.ds(p0, PPW), :],
                    pbuf.at[pl.ds(sbase, PPW), :])
    for h in ih:
        h.wait()
    bufs = [b0, b1, b2]

    def init_pe(ci):
        return pltpu.async_copy(
            pbuf.at[pl.ds(sbase + (ci % QPB) * C, C), :],
            bufs[ci % NBUF], psem)

    ph = [init_pe(0), init_pe(1)]
    gh = []
    oh = []
    for ci in range(NCHUNK):
        if ci + 2 < NCHUNK:
            if ci + 2 - NBUF >= 0:
                oh[ci + 2 - NBUF].wait()
            ph.append(init_pe(ci + 2))
        if ci == 0:
            ph[0].wait()
            gh.append(pltpu.async_copy(
                table_hbm.at[idx_v.at[pl.ds(0, C)]], bufs[0], gsem, add=True))
        if ci + 1 < NCHUNK:
            ph[ci + 1].wait()
            gh.append(pltpu.async_copy(
                table_hbm.at[idx_v.at[pl.ds((ci + 1) * C, C)]],
                bufs[(ci + 1) % NBUF], gsem, add=True))
        gh[ci].wait()
        b = ci // QPB
        oh.append(pltpu.async_copy(
            bufs[ci % NBUF],
            out_hbm.at[pl.ds(b * NTOK + p0 + (ci % QPB) * C, C), :], osem))
    for hh in oh[NCHUNK - NBUF + 2:]:
        hh.wait()


def kernel(x, embed_weight, positional_encoding):
    idx = x.reshape(-1).astype(jnp.int32)
    out = _emb_kernel(idx, embed_weight, positional_encoding)
    return out.reshape(x.shape[0], x.shape[1], D)


# speedup vs baseline: 1.1644x; 1.0306x over previous
import functools

import jax
import jax.numpy as jnp
from jax import lax
from jax.experimental import pallas as pl
from jax.experimental.pallas import tpu as pltpu
from jax.experimental.pallas import tpu_sc as plsc

D = 768
NTOK = 2048
BATCH = 4
B = BATCH * NTOK

NC = 2
NS = 16
NW = NC * NS
PPW = NTOK // NW
C = 16
NCHUNK = BATCH * PPW // C
QPB = PPW // C
NBUF = 6

_mesh = plsc.VectorSubcoreMesh(core_axis_name="c", subcore_axis_name="s")


@functools.partial(
    pl.kernel,
    mesh=_mesh,
    out_type=jax.ShapeDtypeStruct((B, D), jnp.float32),
    scratch_types=[
        pltpu.VMEM((BATCH * PPW,), jnp.int32),
        pltpu.VMEM_SHARED((NS * PPW, D), jnp.float32),
    ] + [pltpu.VMEM((C, D), jnp.float32) for _ in range(NBUF)] + [
        pltpu.SemaphoreType.DMA,
        pltpu.SemaphoreType.DMA,
        pltpu.SemaphoreType.DMA,
        pltpu.SemaphoreType.DMA,
        pltpu.SemaphoreType.DMA,
    ],
)
def _emb_kernel(idx_hbm, table_hbm, pe_hbm, out_hbm,
                idx_v, pbuf, b0, b1, b2, b3, b4, b5,
                gsem, psem, osem, isem, ssem):
    sid = lax.axis_index("s")
    wid = sid * NC + lax.axis_index("c")
    p0 = wid * PPW
    sbase = sid * PPW
    bufs = [b0, b1, b2, b3, b4, b5]

    ih = [pltpu.async_copy(idx_hbm.at[pl.ds(b * NTOK + p0, PPW)],
                           idx_v.at[pl.ds(b * PPW, PPW)], isem)
          for b in range(BATCH)]
    sh = pltpu.async_copy(pe_hbm.at[pl.ds(p0, PPW), :],
                          pbuf.at[pl.ds(sbase, PPW), :], ssem)

    def pe_rows(ci):
        return (ci % QPB) * C

    ph = [pltpu.async_copy(pe_hbm.at[pl.ds(p0 + pe_rows(ci), C), :],
                           bufs[ci], psem)
          for ci in range(3)]

    idx_waited = set()

    def gather_add(ci):
        seg = (ci * C) // PPW
        if seg not in idx_waited:
            ih[seg].wait()
            idx_waited.add(seg)
        return pltpu.async_copy(
            table_hbm.at[idx_v.at[pl.ds(ci * C, C)]],
            bufs[ci % NBUF], gsem, add=True)

    ph[0].wait()
    gh = [gather_add(0)]
    ph[1].wait()
    gh.append(gather_add(1))
    stage_done = False
    oh = []
    for ci in range(NCHUNK):
        if ci + 3 < NCHUNK:
            if ci - 3 >= 0:
                oh[ci - 3].wait()
            if not stage_done:
                sh.wait()
                stage_done = True
            ph.append(pltpu.async_copy(
                pbuf.at[pl.ds(sbase + pe_rows(ci + 3), C), :],
                bufs[(ci + 3) % NBUF], psem))
        if ci + 2 < NCHUNK:
            ph[ci + 2].wait()
            gh.append(gather_add(ci + 2))
        gh[ci].wait()
        b = ci // QPB
        oh.append(pltpu.async_copy(
            bufs[ci % NBUF],
            out_hbm.at[pl.ds(b * NTOK + p0 + pe_rows(ci), C), :], osem))
    for hh in oh[NCHUNK - 6:]:
        hh.wait()


def kernel(x, embed_weight, positional_encoding):
    idx = x.reshape(-1).astype(jnp.int32)
    out = _emb_kernel(idx, embed_weight, positional_encoding)
    return out.reshape(x.shape[0], x.shape[1], D)


# speedup vs baseline: 1.1645x; 1.0001x over previous
import functools

import jax
import jax.numpy as jnp
from jax import lax
from jax.experimental import pallas as pl
from jax.experimental.pallas import tpu as pltpu
from jax.experimental.pallas import tpu_sc as plsc

D = 768
NTOK = 2048
BATCH = 4
B = BATCH * NTOK

NC = 2
NS = 16
NW = NC * NS
PPW = NTOK // NW
C = 16
NCHUNK = BATCH * PPW // C
QPB = PPW // C
NBUF = 6

_mesh = plsc.VectorSubcoreMesh(core_axis_name="c", subcore_axis_name="s")


@functools.partial(
    pl.kernel,
    mesh=_mesh,
    out_type=jax.ShapeDtypeStruct((B, D), jnp.float32),
    scratch_types=[
        pltpu.VMEM((BATCH * PPW,), jnp.int32),
        pltpu.VMEM_SHARED((NS * PPW, D), jnp.float32),
    ] + [pltpu.VMEM((C, D), jnp.float32) for _ in range(NBUF)] + [
        pltpu.SemaphoreType.DMA,
        pltpu.SemaphoreType.DMA,
        pltpu.SemaphoreType.DMA,
        pltpu.SemaphoreType.DMA,
        pltpu.SemaphoreType.DMA,
    ],
)
def _emb_kernel(idx_hbm, table_hbm, pe_hbm, out_hbm,
                idx_v, pbuf, b0, b1, b2, b3, b4, b5,
                gsem, psem, osem, isem, ssem):
    sid = lax.axis_index("s")
    wid = sid * NC + lax.axis_index("c")
    p0 = wid * PPW
    sbase = sid * PPW
    bufs = [b0, b1, b2, b3, b4, b5]

    ih = [pltpu.async_copy(idx_hbm.at[pl.ds(b * NTOK + p0, PPW)],
                           idx_v.at[pl.ds(b * PPW, PPW)], isem)
          for b in range(BATCH)]
    sh = pltpu.async_copy(pe_hbm.at[pl.ds(p0, PPW), :],
                          pbuf.at[pl.ds(sbase, PPW), :], ssem)

    def pe_rows(ci):
        return (ci % QPB) * C

    ph = [pltpu.async_copy(pe_hbm.at[pl.ds(p0 + pe_rows(ci), C), :],
                           bufs[ci], psem)
          for ci in range(4)]

    idx_waited = set()

    def gather_add(ci):
        seg = (ci * C) // PPW
        if seg not in idx_waited:
            ih[seg].wait()
            idx_waited.add(seg)
        return pltpu.async_copy(
            table_hbm.at[idx_v.at[pl.ds(ci * C, C)]],
            bufs[ci % NBUF], gsem, add=True)

    gh = []
    for k in range(3):
        ph[k].wait()
        gh.append(gather_add(k))
    stage_done = False
    oh = []
    for ci in range(NCHUNK):
        if ci + 4 < NCHUNK:
            if ci - 2 >= 0:
                oh[ci - 2].wait()
            if not stage_done:
                sh.wait()
                stage_done = True
            ph.append(pltpu.async_copy(
                pbuf.at[pl.ds(sbase + pe_rows(ci + 4), C), :],
                bufs[(ci + 4) % NBUF], psem))
        if ci + 3 < NCHUNK:
            ph[ci + 3].wait()
            gh.append(gather_add(ci + 3))
        gh[ci].wait()
        b = ci // QPB
        oh.append(pltpu.async_copy(
            bufs[ci % NBUF],
            out_hbm.at[pl.ds(b * NTOK + p0 + pe_rows(ci), C), :], osem))
    for hh in oh[NCHUNK - 6:]:
        hh.wait()


def kernel(x, embed_weight, positional_encoding):
    idx = x.reshape(-1).astype(jnp.int32)
    out = _emb_kernel(idx, embed_weight, positional_encoding)
    return out.reshape(x.shape[0], x.shape[1], D)


# speedup vs baseline: 1.1729x; 1.0072x over previous
import functools

import jax
import jax.numpy as jnp
from jax import lax
from jax.experimental import pallas as pl
from jax.experimental.pallas import tpu as pltpu
from jax.experimental.pallas import tpu_sc as plsc

D = 768
NTOK = 2048
BATCH = 4
B = BATCH * NTOK

NC = 2
NS = 16
NW = NC * NS
PPW = NTOK // NW
C = 16
NCHUNK = BATCH * PPW // C
QPB = PPW // C
NBUF = 6

_mesh = plsc.VectorSubcoreMesh(core_axis_name="c", subcore_axis_name="s")


@functools.partial(
    pl.kernel,
    mesh=_mesh,
    out_type=jax.ShapeDtypeStruct((B, D), jnp.float32),
    scratch_types=[
        pltpu.VMEM((BATCH * PPW,), jnp.int32),
        pltpu.VMEM_SHARED((NS * PPW, D), jnp.float32),
    ] + [pltpu.VMEM((C, D), jnp.float32) for _ in range(NBUF)] + [
        pltpu.SemaphoreType.DMA,
        pltpu.SemaphoreType.DMA,
        pltpu.SemaphoreType.DMA,
        pltpu.SemaphoreType.DMA,
        pltpu.SemaphoreType.DMA,
    ],
)
def _emb_kernel(idx_hbm, table_hbm, pe_hbm, out_hbm,
                idx_v, pbuf, b0, b1, b2, b3, b4, b5,
                gsem, psem, osem, isem, ssem):
    sid = lax.axis_index("s")
    wid = sid * NC + lax.axis_index("c")
    p0 = wid * PPW
    sbase = sid * PPW
    bufs = [b0, b1, b2, b3, b4, b5]

    ih = [pltpu.async_copy(idx_hbm.at[b, pl.ds(p0, PPW)],
                           idx_v.at[pl.ds(b * PPW, PPW)], isem)
          for b in range(BATCH)]
    sh = pltpu.async_copy(pe_hbm.at[pl.ds(p0, PPW), :],
                          pbuf.at[pl.ds(sbase, PPW), :], ssem)

    def pe_rows(ci):
        return (ci % QPB) * C

    ph = [pltpu.async_copy(pe_hbm.at[pl.ds(p0 + pe_rows(ci), C), :],
                           bufs[ci], psem)
          for ci in range(4)]

    idx_waited = set()

    def gather_add(ci):
        seg = (ci * C) // PPW
        if seg not in idx_waited:
            ih[seg].wait()
            idx_waited.add(seg)
        return pltpu.async_copy(
            table_hbm.at[idx_v.at[pl.ds(ci * C, C)]],
            bufs[ci % NBUF], gsem, add=True)

    gh = []
    for k in range(3):
        ph[k].wait()
        gh.append(gather_add(k))
    stage_done = False
    oh = []
    for ci in range(NCHUNK):
        if ci + 4 < NCHUNK:
            if ci - 2 >= 0:
                oh[ci - 2].wait()
            if not stage_done:
                sh.wait()
                stage_done = True
            ph.append(pltpu.async_copy(
                pbuf.at[pl.ds(sbase + pe_rows(ci + 4), C), :],
                bufs[(ci + 4) % NBUF], psem))
        if ci + 3 < NCHUNK:
            ph[ci + 3].wait()
            gh.append(gather_add(ci + 3))
        gh[ci].wait()
        b = ci // QPB
        oh.append(pltpu.async_copy(
            bufs[ci % NBUF],
            out_hbm.at[pl.ds(b * NTOK + p0 + pe_rows(ci), C), :], osem))
    for hh in oh[NCHUNK - 6:]:
        hh.wait()


def kernel(x, embed_weight, positional_encoding):
    out = _emb_kernel(x.astype(jnp.int32), embed_weight, positional_encoding)
    return out.reshape(x.shape[0], x.shape[1], D)
